# Initial kernel scaffold; baseline (speedup 1.0000x reference)
import jax
import jax.numpy as jnp
from jax.experimental import pallas as pl


def kernel(x, edge_index, pos, Wd, bd, W1, as1, ad1, b1, W2, as2, ad2, b2):
    raise NotImplementedError("write your pallas kernel here")


# speedup vs baseline: 32.9086x; 32.9086x over previous
import functools

import jax
import jax.numpy as jnp
from jax import lax
from jax.experimental import pallas as pl
from jax.experimental.pallas import tpu as pltpu
from jax.experimental.pallas import tpu_sc as plsc

N = 10000
E = 320000
D = 128
H = 8
C = 16

NC = 2
NS = 16
NW = NC * NS
EPT = E // NW
CH = 80
NCHUNK = EPT // CH

BLK = 400
GRID = N // BLK

_f32 = jnp.float32
_i32 = jnp.int32
_HI = jax.lax.Precision.HIGHEST


def _tc_dense_body(x_ref, w_ref, pa_ref, pd_ref,
                   h_ref, av_ref, dv_ref, mav_ref, mdv_ref):
    i = pl.program_id(0)
    h = jnp.dot(x_ref[...], w_ref[...], preferred_element_type=_f32)
    h_ref[...] = h
    av = jnp.dot(h, pa_ref[...], preferred_element_type=_f32, precision=_HI)
    dv = jnp.dot(h, pd_ref[...], preferred_element_type=_f32, precision=_HI)
    av_ref[...] = av
    dv_ref[...] = dv

    @pl.when(i == 0)
    def _():
        mav_ref[...] = jnp.full((1, D), -3e38, _f32)
        mdv_ref[...] = jnp.full((1, D), -3e38, _f32)

    mav_ref[...] = jnp.maximum(mav_ref[...], jnp.max(av, axis=0, keepdims=True))
    mdv_ref[...] = jnp.maximum(mdv_ref[...], jnp.max(dv, axis=0, keepdims=True))


def _tc_dense(x, w, pa, pd):
    return pl.pallas_call(
        _tc_dense_body,
        grid=(GRID,),
        in_specs=[
            pl.BlockSpec((BLK, D), lambda i: (i, 0)),
            pl.BlockSpec((D, D), lambda i: (0, 0)),
            pl.BlockSpec((D, D), lambda i: (0, 0)),
            pl.BlockSpec((D, D), lambda i: (0, 0)),
        ],
        out_specs=[
            pl.BlockSpec((BLK, D), lambda i: (i, 0)),
            pl.BlockSpec((BLK, D), lambda i: (i, 0)),
            pl.BlockSpec((BLK, D), lambda i: (i, 0)),
            pl.BlockSpec((1, D), lambda i: (0, 0)),
            pl.BlockSpec((1, D), lambda i: (0, 0)),
        ],
        out_shape=[
            jax.ShapeDtypeStruct((N, D), _f32),
            jax.ShapeDtypeStruct((N, D), _f32),
            jax.ShapeDtypeStruct((N, D), _f32),
            jax.ShapeDtypeStruct((1, D), _f32),
            jax.ShapeDtypeStruct((1, D), _f32),
        ],
        compiler_params=pltpu.CompilerParams(
            dimension_semantics=("arbitrary",)),
    )(x, w, pa, pd)


def _tc_dvrd_body(d0_ref, d1_ref, dv_ref, o_ref):
    rd = 1.0 / (d0_ref[...][:, :H] + d1_ref[...][:, :H] + 1e-16)
    o_ref[...] = jnp.concatenate(
        [dv_ref[...][:, :H], rd, jnp.zeros((BLK, D - 2 * H), _f32)], axis=1)


def _tc_dvrd(d0, d1, dv):
    return pl.pallas_call(
        _tc_dvrd_body,
        grid=(GRID,),
        in_specs=[
            pl.BlockSpec((BLK, D), lambda i: (i, 0)),
            pl.BlockSpec((BLK, D), lambda i: (i, 0)),
            pl.BlockSpec((BLK, D), lambda i: (i, 0)),
        ],
        out_specs=pl.BlockSpec((BLK, D), lambda i: (i, 0)),
        out_shape=jax.ShapeDtypeStruct((N, D), _f32),
        compiler_params=pltpu.CompilerParams(
            dimension_semantics=("arbitrary",)),
    )(d0, d1, dv)


def _tc_mid_body(a0_ref, a1_ref, b_ref, w_ref, pa_ref, pd_ref,
                 hp_ref, h_ref, av_ref, dv_ref, mav_ref, mdv_ref):
    i = pl.program_id(0)
    hp = jnp.maximum(a0_ref[...] + a1_ref[...] + b_ref[...], 0.0)
    hp_ref[...] = hp
    h = jnp.dot(hp, w_ref[...], preferred_element_type=_f32)
    h_ref[...] = h
    av = jnp.dot(h, pa_ref[...], preferred_element_type=_f32, precision=_HI)
    dv = jnp.dot(h, pd_ref[...], preferred_element_type=_f32, precision=_HI)
    av_ref[...] = av
    dv_ref[...] = dv

    @pl.when(i == 0)
    def _():
        mav_ref[...] = jnp.full((1, D), -3e38, _f32)
        mdv_ref[...] = jnp.full((1, D), -3e38, _f32)

    mav_ref[...] = jnp.maximum(mav_ref[...], jnp.max(av, axis=0, keepdims=True))
    mdv_ref[...] = jnp.maximum(mdv_ref[...], jnp.max(dv, axis=0, keepdims=True))


def _tc_mid(a0, a1, b, w, pa, pd):
    return pl.pallas_call(
        _tc_mid_body,
        grid=(GRID,),
        in_specs=[
            pl.BlockSpec((BLK, D), lambda i: (i, 0)),
            pl.BlockSpec((BLK, D), lambda i: (i, 0)),
            pl.BlockSpec((1, D), lambda i: (0, 0)),
            pl.BlockSpec((D, D), lambda i: (0, 0)),
            pl.BlockSpec((D, D), lambda i: (0, 0)),
            pl.BlockSpec((D, D), lambda i: (0, 0)),
        ],
        out_specs=[
            pl.BlockSpec((BLK, D), lambda i: (i, 0)),
            pl.BlockSpec((BLK, D), lambda i: (i, 0)),
            pl.BlockSpec((BLK, D), lambda i: (i, 0)),
            pl.BlockSpec((BLK, D), lambda i: (i, 0)),
            pl.BlockSpec((1, D), lambda i: (0, 0)),
            pl.BlockSpec((1, D), lambda i: (0, 0)),
        ],
        out_shape=[
            jax.ShapeDtypeStruct((N, D), _f32),
            jax.ShapeDtypeStruct((N, D), _f32),
            jax.ShapeDtypeStruct((N, D), _f32),
            jax.ShapeDtypeStruct((N, D), _f32),
            jax.ShapeDtypeStruct((1, D), _f32),
            jax.ShapeDtypeStruct((1, D), _f32),
        ],
        compiler_params=pltpu.CompilerParams(
            dimension_semantics=("arbitrary",)),
    )(a0, a1, b, w, pa, pd)


def _tc_final_body(x_ref, hp_ref, a0_ref, a1_ref, b_ref, o_ref):
    h2 = jnp.maximum(a0_ref[...] + a1_ref[...] + b_ref[...], 0.0)
    o_ref[...] = x_ref[...] + hp_ref[...] + h2


def _tc_final(x, hp, a0, a1, b):
    return pl.pallas_call(
        _tc_final_body,
        grid=(GRID,),
        in_specs=[
            pl.BlockSpec((BLK, D), lambda i: (i, 0)),
            pl.BlockSpec((BLK, D), lambda i: (i, 0)),
            pl.BlockSpec((BLK, D), lambda i: (i, 0)),
            pl.BlockSpec((BLK, D), lambda i: (i, 0)),
            pl.BlockSpec((1, D), lambda i: (0, 0)),
        ],
        out_specs=pl.BlockSpec((BLK, D), lambda i: (i, 0)),
        out_shape=jax.ShapeDtypeStruct((N, D), _f32),
        compiler_params=pltpu.CompilerParams(
            dimension_semantics=("arbitrary",)),
    )(x, hp, a0, a1, b)


RPS = 624
RLAST = N - (NS - 1) * RPS


def _stage_rows(body_fn):
    s = lax.axis_index("s")

    @pl.when(s < NS - 1)
    def _():
        body_fn(pl.multiple_of(s * RPS, 8), RPS)

    @pl.when(s == NS - 1)
    def _():
        body_fn(pl.multiple_of((NS - 1) * RPS, 8), RLAST)


def _sc_denom_kernel(src_h, dst_h, av_h, dv_h, msub_h, z_h, dpart_h,
                     dacc, sidx, didx, avb, dvb, exb, msub_v):
    c = lax.axis_index("c")
    s = lax.axis_index("s")
    wid = c * NS + s

    _stage_rows(lambda r0, nr: pltpu.sync_copy(
        z_h.at[pl.ds(r0, nr), :], dacc.at[pl.ds(r0, nr), :]))
    pltpu.sync_copy(z_h.at[pl.ds(0, CH), :], exb)
    pltpu.sync_copy(msub_h, msub_v)
    plsc.subcore_barrier()

    m16 = msub_v[...]
    base = wid * EPT

    @pl.loop(0, NCHUNK)
    def _chunk(k):
        e0 = pl.multiple_of(base + k * CH, 8)
        pltpu.sync_copy(src_h.at[pl.ds(e0, CH)], sidx)
        pltpu.sync_copy(dst_h.at[pl.ds(e0, CH)], didx)
        pltpu.sync_copy(av_h.at[sidx], avb)
        pltpu.sync_copy(dv_h.at[didx], dvb)

        @pl.loop(0, CH)
        def _e(e):
            t = avb[e, pl.ds(0, 16)] + dvb[e, pl.ds(0, 16)]
            t = jnp.where(t > 0.0, t, 0.2 * t)
            exb[e, pl.ds(0, 16)] = jnp.exp(t - m16)

        pltpu.sync_copy(exb, dacc.at[didx], add=True)

    plsc.subcore_barrier()
    _stage_rows(lambda r0, nr: pltpu.sync_copy(
        dacc.at[pl.ds(r0, nr), :], dpart_h.at[c, pl.ds(r0, nr), :]))


def _sc_denom(src, dst, av, dv, msub, z):
    mesh = plsc.VectorSubcoreMesh(core_axis_name="c", subcore_axis_name="s")
    kern = functools.partial(
        pl.kernel,
        out_type=jax.ShapeDtypeStruct((NC, N, D), _f32),
        mesh=mesh,
        scratch_types=[
            pltpu.VMEM_SHARED((N, D), _f32),
            pltpu.VMEM((CH,), _i32),
            pltpu.VMEM((CH,), _i32),
            pltpu.VMEM((CH, D), _f32),
            pltpu.VMEM((CH, D), _f32),
            pltpu.VMEM((CH, D), _f32),
            pltpu.VMEM((16,), _f32),
        ],
    )(_sc_denom_kernel)
    return kern(src, dst, av, dv, msub, z)


def _sc_msg_kernel(src_h, dst_h, av_h, dvrd_h, msub_h, h_h, z_h,
                   aggp_h,
                   acc, sidx, didx, avb, ddb, hb, msub_v):
    c = lax.axis_index("c")
    s = lax.axis_index("s")
    wid = c * NS + s

    _stage_rows(lambda r0, nr: pltpu.sync_copy(
        z_h.at[pl.ds(r0, nr), :], acc.at[pl.ds(r0, nr), :]))
    pltpu.sync_copy(msub_h, msub_v)
    plsc.subcore_barrier()

    m16 = msub_v[...]
    base = wid * EPT

    @pl.loop(0, NCHUNK)
    def _chunk(k):
        e0 = pl.multiple_of(base + k * CH, 8)
        pltpu.sync_copy(src_h.at[pl.ds(e0, CH)], sidx)
        pltpu.sync_copy(dst_h.at[pl.ds(e0, CH)], didx)
        pltpu.sync_copy(h_h.at[sidx], hb)
        pltpu.sync_copy(av_h.at[sidx], avb)
        pltpu.sync_copy(dvrd_h.at[didx], ddb)

        @pl.loop(0, CH)
        def _e(e):
            dd = ddb[e, pl.ds(0, 16)]
            u = avb[e, pl.ds(0, 16)] + dd
            t = jnp.where(u > 0.0, u, 0.2 * u)
            ex = jnp.exp(t - m16)
            for hi in range(H):
                hv = hb[e, pl.ds(hi * 16, 16)]
                hb[e, pl.ds(hi * 16, 16)] = hv * (ex[hi] * dd[H + hi])

        pltpu.sync_copy(hb, acc.at[didx], add=True)

    plsc.subcore_barrier()
    _stage_rows(lambda r0, nr: pltpu.sync_copy(
        acc.at[pl.ds(r0, nr), :], aggp_h.at[c, pl.ds(r0, nr), :]))


def _sc_msg(src, dst, av, dvrd, msub, h, z):
    mesh = plsc.VectorSubcoreMesh(core_axis_name="c", subcore_axis_name="s")
    kern = functools.partial(
        pl.kernel,
        out_type=jax.ShapeDtypeStruct((NC, N, D), _f32),
        mesh=mesh,
        scratch_types=[
            pltpu.VMEM_SHARED((N, D), _f32),
            pltpu.VMEM((CH,), _i32),
            pltpu.VMEM((CH,), _i32),
            pltpu.VMEM((CH, D), _f32),
            pltpu.VMEM((CH, D), _f32),
            pltpu.VMEM((CH, D), _f32),
            pltpu.VMEM((16,), _f32),
        ],
    )(_sc_msg_kernel)
    return kern(src, dst, av, dvrd, msub, h, z)


def _head_proj(att):
    d = jnp.arange(D)
    hh = jnp.arange(D) % H
    mask = ((d[:, None] // C == hh[None, :]) &
            (jnp.arange(D)[None, :] < 16)).astype(_f32)
    return mask * att.reshape(-1)[:, None]


def _mshift(mav, mdv):
    m = mav[0, :16] + mdv[0, :16]
    return jnp.where(m > 0.0, m, 0.2 * m)


def kernel(x, edge_index, pos, Wd, bd, W1, as1, ad1, b1, W2, as2, ad2, b2):
    del pos, Wd, bd
    src = edge_index[0]
    dst = edge_index[1]
    z = jnp.zeros((N, D), _f32)

    pa1, pd1 = _head_proj(as1), _head_proj(ad1)
    pa2, pd2 = _head_proj(as2), _head_proj(ad2)

    h1, av1, dv1, mav1, mdv1 = _tc_dense(x, W1, pa1, pd1)
    msub1 = _mshift(mav1, mdv1)
    d1p = _sc_denom(src, dst, av1, dv1, msub1, z)
    dvrd1 = _tc_dvrd(d1p[0], d1p[1], dv1)
    a1p = _sc_msg(src, dst, av1, dvrd1, msub1, h1, z)

    hpost, h2, av2, dv2, mav2, mdv2 = _tc_mid(
        a1p[0], a1p[1], b1.reshape(1, D), W2, pa2, pd2)
    msub2 = _mshift(mav2, mdv2)
    d2p = _sc_denom(src, dst, av2, dv2, msub2, z)
    dvrd2 = _tc_dvrd(d2p[0], d2p[1], dv2)
    a2p = _sc_msg(src, dst, av2, dvrd2, msub2, h2, z)

    return _tc_final(x, hpost, a2p[0], a2p[1], b2.reshape(1, D))
